# baseline (speedup 1.0000x reference)
import jax
import jax.numpy as jnp
from jax.experimental import pallas as pl

NUM_NODES = 207
RNN_UNITS = 64
K_DIFF = 2
NUM_CLASSES = 5
INPUT_DIM = 2
BATCH = 32
SEQ_LEN = 12
_PREC = jax.lax.Precision.DEFAULT
NM = K_DIFF + 1
NP = BATCH // 2
NR = NP * NUM_NODES
HL = 2 * RNN_UNITS


def _split_xh(W, in_dim):
    out = W.shape[1]
    W3 = jnp.transpose(W.reshape(in_dim + RNN_UNITS, NM, out), (1, 0, 2))
    return W3[:, :in_dim, :], W3[:, in_dim:, :]


def _dup_gate(Wb):
    km, f, _ = Wb.shape
    W4 = Wb.reshape(km, f, 2, RNN_UNITS)
    eye = jnp.eye(2, dtype=Wb.dtype)
    W6 = jnp.einsum('kfgu,ab->kafgbu', W4, eye)
    return W6.reshape(km * 2 * f, 4 * RNN_UNITS)


def _dup_cand(Wb):
    km, f, _ = Wb.shape
    eye = jnp.eye(2, dtype=Wb.dtype)
    W5 = jnp.einsum('kfu,ab->kafbu', Wb, eye)
    return W5.reshape(km * 2 * f, 2 * RNN_UNITS)


def _cheb(S, x):
    c = x.shape[1]
    x3 = x.reshape(NP, NUM_NODES, c)
    y1 = jax.lax.dot_general(
        S, x3, (((2,), (1,)), ((0,), (0,))),
        preferred_element_type=jnp.float32, precision=_PREC)
    y2 = 2.0 * jax.lax.dot_general(
        S, y1, (((2,), (1,)), ((0,), (0,))),
        preferred_element_type=jnp.float32, precision=_PREC) - x3
    return x, y1.reshape(NR, c), y2.reshape(NR, c)


def _cell(S, xcat, Wxg, Wxc, h, Whg, bg, Whc, bc):
    h0, h1, h2 = _cheb(S, h)
    hcat = jnp.concatenate([h0, h1, h2], axis=1)
    pre_g = (jnp.dot(xcat, Wxg, preferred_element_type=jnp.float32, precision=_PREC)
             + jnp.dot(hcat, Whg, preferred_element_type=jnp.float32, precision=_PREC) + bg)
    gates = jax.nn.sigmoid(pre_g)
    r = gates[:, :HL]
    u = gates[:, HL:]
    r0, r1, r2 = _cheb(S, r * h)
    rcat = jnp.concatenate([r0, r1, r2], axis=1)
    c = jnp.tanh(jnp.dot(xcat, Wxc, preferred_element_type=jnp.float32, precision=_PREC)
                 + jnp.dot(rcat, Whc, preferred_element_type=jnp.float32, precision=_PREC) + bc)
    return u * h + (1.0 - u) * c


def _body(x_ref, s_ref, wxg0_ref, wxc0_ref, whg0_ref, whc0_ref, bg0_ref,
          bc0_ref, wg1_ref, wc1_ref, bg1_ref, bc1_ref, wfc_ref, bfc_ref,
          mask_ref, out_ref):
    S = jnp.broadcast_to(s_ref[...], (NP, NUM_NODES, NUM_NODES))
    wxg0 = wxg0_ref[...]
    wxc0 = wxc0_ref[...]
    whg0 = whg0_ref[...]
    whc0 = whc0_ref[...]
    bg0 = bg0_ref[...]
    bc0 = bc0_ref[...]
    wg1 = wg1_ref[...]
    wc1 = wc1_ref[...]
    bg1 = bg1_ref[...]
    bc1 = bc1_ref[...]

    def step(t, carry):
        h0, h1, last = carry
        xa, xb, xc = _cheb(S, x_ref[t].T)
        xcat0 = jnp.concatenate([xa, xb, xc], axis=1)
        h0 = _cell(S, xcat0, wxg0, wxc0, h0, whg0, bg0, whc0, bc0)
        ya, yb, yc = _cheb(S, h0)
        xcat1 = jnp.concatenate([ya, yb, yc], axis=1)
        h1 = _cell(S, xcat1, wg1[:3 * HL], wc1[:3 * HL], h1,
                   wg1[3 * HL:], bg1, wc1[3 * HL:], bc1)
        m = mask_ref[t].reshape(NP, 1, HL)
        last = last + h1.reshape(NP, NUM_NODES, HL) * m
        return h0, h1, last

    h0 = jnp.zeros((NR, HL), jnp.float32)
    h1 = jnp.zeros((NR, HL), jnp.float32)
    last = jnp.zeros((NP, NUM_NODES, HL), jnp.float32)
    h0, h1, last = jax.lax.fori_loop(0, SEQ_LEN, step, (h0, h1, last))

    lr = jax.nn.relu(last.reshape(NR, HL))
    logits = (jnp.dot(lr, wfc_ref[...], preferred_element_type=jnp.float32, precision=_PREC)
              + bfc_ref[...])
    pooled = jnp.max(logits.reshape(NP, NUM_NODES, 2 * NUM_CLASSES), axis=1)
    out_ref[...] = pooled


def kernel(input_seq, seq_lengths, supports, Wg0, bg0, Wc0, bc0,
           Wg1, bg1, Wc1, bc1, W_fc, b_fc):
    xseq = jnp.transpose(
        input_seq.reshape(NP, 2, SEQ_LEN, NUM_NODES, INPUT_DIM),
        (2, 1, 4, 0, 3)).reshape(SEQ_LEN, 2 * INPUT_DIM, NR)
    S = supports[0]

    wx0g, wh0g = _split_xh(Wg0, INPUT_DIM)
    wx0c, wh0c = _split_xh(Wc0, INPUT_DIM)
    wx1g, wh1g = _split_xh(Wg1, RNN_UNITS)
    wx1c, wh1c = _split_xh(Wc1, RNN_UNITS)
    wxg0 = _dup_gate(wx0g)
    wxc0 = _dup_cand(wx0c)
    whg0 = _dup_gate(wh0g)
    whc0 = _dup_cand(wh0c)
    wg1 = jnp.concatenate([_dup_gate(wx1g), _dup_gate(wh1g)], axis=0)
    wc1 = jnp.concatenate([_dup_cand(wx1c), _dup_cand(wh1c)], axis=0)

    bg = jnp.broadcast_to(bg0.reshape(2, 1, RNN_UNITS),
                          (2, 2, RNN_UNITS)).reshape(1, 4 * RNN_UNITS)
    bc = jnp.broadcast_to(bc0.reshape(1, RNN_UNITS),
                          (2, RNN_UNITS)).reshape(1, 2 * RNN_UNITS)
    bg1r = jnp.broadcast_to(bg1.reshape(2, 1, RNN_UNITS),
                            (2, 2, RNN_UNITS)).reshape(1, 4 * RNN_UNITS)
    bc1r = jnp.broadcast_to(bc1.reshape(1, RNN_UNITS),
                            (2, RNN_UNITS)).reshape(1, 2 * RNN_UNITS)

    eye = jnp.eye(2, dtype=W_fc.dtype)
    wfc2 = jnp.einsum('uc,ab->aubc', W_fc, eye).reshape(
        2 * RNN_UNITS, 2 * NUM_CLASSES)
    bfc2 = jnp.broadcast_to(b_fc.reshape(1, NUM_CLASSES),
                            (2, NUM_CLASSES)).reshape(1, 2 * NUM_CLASSES)

    idx = jnp.clip(seq_lengths - 1, 0, SEQ_LEN - 1)
    onehot = (jnp.arange(SEQ_LEN)[:, None] == idx[None, :]).astype(jnp.float32)
    mask = jnp.repeat(onehot.reshape(SEQ_LEN, NP, 2, 1), RNN_UNITS,
                      axis=3).reshape(SEQ_LEN, NP, HL)

    pooled2 = pl.pallas_call(
        _body,
        out_shape=jax.ShapeDtypeStruct((NP, 2 * NUM_CLASSES), jnp.float32),
    )(xseq, S, wxg0, wxc0, whg0, whc0, bg, bc, wg1, wc1, bg1r, bc1r,
      wfc2, bfc2, mask)
    return pooled2.reshape(BATCH, NUM_CLASSES)


# speedup vs baseline: 2.1653x; 2.1653x over previous
import jax
import jax.numpy as jnp
from jax.experimental import pallas as pl

NUM_NODES = 207
RNN_UNITS = 64
K_DIFF = 2
NUM_CLASSES = 5
INPUT_DIM = 2
BATCH = 32
SEQ_LEN = 12
_PREC = jax.lax.Precision.DEFAULT
NM = K_DIFF + 1
NP = BATCH // 2
NN = 208
NR = NP * NN
HL = 2 * RNN_UNITS


def _split_xh(W, in_dim):
    out = W.shape[1]
    W3 = jnp.transpose(W.reshape(in_dim + RNN_UNITS, NM, out), (1, 0, 2))
    return W3[:, :in_dim, :], W3[:, in_dim:, :]


def _dup_gate(Wb):
    km, f, _ = Wb.shape
    W4 = Wb.reshape(km, f, 2, RNN_UNITS)
    eye = jnp.eye(2, dtype=Wb.dtype)
    W6 = jnp.einsum('kfgu,ab->kafgbu', W4, eye)
    return W6.reshape(km * 2 * f, 4 * RNN_UNITS)


def _dup_cand(Wb):
    km, f, _ = Wb.shape
    eye = jnp.eye(2, dtype=Wb.dtype)
    W5 = jnp.einsum('kfu,ab->kafbu', Wb, eye)
    return W5.reshape(km * 2 * f, 2 * RNN_UNITS)


def _cheb(S, x):
    c = x.shape[1]
    x3 = x.reshape(NP, NN, c)
    y1 = jax.lax.dot_general(
        S, x3, (((2,), (1,)), ((0,), (0,))),
        preferred_element_type=jnp.float32, precision=_PREC)
    y2 = 2.0 * jax.lax.dot_general(
        S, y1, (((2,), (1,)), ((0,), (0,))),
        preferred_element_type=jnp.float32, precision=_PREC) - x3
    return x, y1.reshape(NR, c), y2.reshape(NR, c)


def _cell(S, xcat, Wxg, Wxc, h, Whg, bg, Whc, bc):
    h0, h1, h2 = _cheb(S, h)
    hcat = jnp.concatenate([h0, h1, h2], axis=1)
    pre_g = (jnp.dot(xcat, Wxg, preferred_element_type=jnp.float32, precision=_PREC)
             + jnp.dot(hcat, Whg, preferred_element_type=jnp.float32, precision=_PREC) + bg)
    gates = jax.nn.sigmoid(pre_g)
    r = gates[:, :HL]
    u = gates[:, HL:]
    r0, r1, r2 = _cheb(S, r * h)
    rcat = jnp.concatenate([r0, r1, r2], axis=1)
    c = jnp.tanh(jnp.dot(xcat, Wxc, preferred_element_type=jnp.float32, precision=_PREC)
                 + jnp.dot(rcat, Whc, preferred_element_type=jnp.float32, precision=_PREC) + bc)
    return u * h + (1.0 - u) * c


def _body(x_ref, s_ref, wxg0_ref, wxc0_ref, whg0_ref, whc0_ref, bg0_ref,
          bc0_ref, wg1_ref, wc1_ref, bg1_ref, bc1_ref, wfc_ref, bfc_ref,
          mask_ref, out_ref):
    S = jnp.broadcast_to(s_ref[...], (NP, NN, NN))
    wxg0 = wxg0_ref[...]
    wxc0 = wxc0_ref[...]
    whg0 = whg0_ref[...]
    whc0 = whc0_ref[...]
    bg0 = bg0_ref[...]
    bc0 = bc0_ref[...]
    wg1 = wg1_ref[...]
    wc1 = wc1_ref[...]
    bg1 = bg1_ref[...]
    bc1 = bc1_ref[...]

    def step(t, carry):
        h0, h1, last = carry
        xa, xb, xc = _cheb(S, x_ref[t].T)
        xcat0 = jnp.concatenate([xa, xb, xc], axis=1)
        h0 = _cell(S, xcat0, wxg0, wxc0, h0, whg0, bg0, whc0, bc0)
        ya, yb, yc = _cheb(S, h0)
        xcat1 = jnp.concatenate([ya, yb, yc], axis=1)
        h1 = _cell(S, xcat1, wg1[:3 * HL], wc1[:3 * HL], h1,
                   wg1[3 * HL:], bg1, wc1[3 * HL:], bc1)
        m = mask_ref[t].reshape(NP, 1, HL)
        last = last + h1.reshape(NP, NN, HL) * m
        return h0, h1, last

    h0 = jnp.zeros((NR, HL), jnp.float32)
    h1 = jnp.zeros((NR, HL), jnp.float32)
    last = jnp.zeros((NP, NN, HL), jnp.float32)
    h0, h1, last = jax.lax.fori_loop(0, SEQ_LEN, step, (h0, h1, last))

    lr = jax.nn.relu(last.reshape(NR, HL))
    logits = (jnp.dot(lr, wfc_ref[...], preferred_element_type=jnp.float32, precision=_PREC)
              + bfc_ref[...])
    logits3 = logits.reshape(NP, NN, 2 * NUM_CLASSES)[:, :NUM_NODES, :]
    pooled = jnp.max(logits3, axis=1)
    out_ref[...] = pooled


def kernel(input_seq, seq_lengths, supports, Wg0, bg0, Wc0, bc0,
           Wg1, bg1, Wc1, bc1, W_fc, b_fc):
    xseq = jnp.transpose(
        input_seq.reshape(NP, 2, SEQ_LEN, NUM_NODES, INPUT_DIM),
        (2, 1, 4, 0, 3))
    xseq = jnp.pad(xseq, ((0, 0), (0, 0), (0, 0), (0, 0), (0, NN - NUM_NODES))
                   ).reshape(SEQ_LEN, 2 * INPUT_DIM, NR)
    S = jnp.pad(supports[0], ((0, NN - NUM_NODES), (0, NN - NUM_NODES)))

    wx0g, wh0g = _split_xh(Wg0, INPUT_DIM)
    wx0c, wh0c = _split_xh(Wc0, INPUT_DIM)
    wx1g, wh1g = _split_xh(Wg1, RNN_UNITS)
    wx1c, wh1c = _split_xh(Wc1, RNN_UNITS)
    wxg0 = _dup_gate(wx0g)
    wxc0 = _dup_cand(wx0c)
    whg0 = _dup_gate(wh0g)
    whc0 = _dup_cand(wh0c)
    wg1 = jnp.concatenate([_dup_gate(wx1g), _dup_gate(wh1g)], axis=0)
    wc1 = jnp.concatenate([_dup_cand(wx1c), _dup_cand(wh1c)], axis=0)

    bg = jnp.broadcast_to(bg0.reshape(2, 1, RNN_UNITS),
                          (2, 2, RNN_UNITS)).reshape(1, 4 * RNN_UNITS)
    bc = jnp.broadcast_to(bc0.reshape(1, RNN_UNITS),
                          (2, RNN_UNITS)).reshape(1, 2 * RNN_UNITS)
    bg1r = jnp.broadcast_to(bg1.reshape(2, 1, RNN_UNITS),
                            (2, 2, RNN_UNITS)).reshape(1, 4 * RNN_UNITS)
    bc1r = jnp.broadcast_to(bc1.reshape(1, RNN_UNITS),
                            (2, RNN_UNITS)).reshape(1, 2 * RNN_UNITS)

    eye = jnp.eye(2, dtype=W_fc.dtype)
    wfc2 = jnp.einsum('uc,ab->aubc', W_fc, eye).reshape(
        2 * RNN_UNITS, 2 * NUM_CLASSES)
    bfc2 = jnp.broadcast_to(b_fc.reshape(1, NUM_CLASSES),
                            (2, NUM_CLASSES)).reshape(1, 2 * NUM_CLASSES)

    idx = jnp.clip(seq_lengths - 1, 0, SEQ_LEN - 1)
    onehot = (jnp.arange(SEQ_LEN)[:, None] == idx[None, :]).astype(jnp.float32)
    mask = jnp.repeat(onehot.reshape(SEQ_LEN, NP, 2, 1), RNN_UNITS,
                      axis=3).reshape(SEQ_LEN, NP, HL)

    pooled2 = pl.pallas_call(
        _body,
        out_shape=jax.ShapeDtypeStruct((NP, 2 * NUM_CLASSES), jnp.float32),
    )(xseq, S, wxg0, wxc0, whg0, whc0, bg, bc, wg1, wc1, bg1r, bc1r,
      wfc2, bfc2, mask)
    return pooled2.reshape(BATCH, NUM_CLASSES)


# speedup vs baseline: 2.8279x; 1.3060x over previous
import jax
import jax.numpy as jnp
from jax.experimental import pallas as pl

NUM_NODES = 207
RNN_UNITS = 64
K_DIFF = 2
NUM_CLASSES = 5
INPUT_DIM = 2
BATCH = 32
SEQ_LEN = 12
_PREC = jax.lax.Precision.DEFAULT
NM = K_DIFF + 1
NP = BATCH // 2
NN = 208
NR = NP * NN
HL = 2 * RNN_UNITS
GL = 2 * HL


def _split_xh(W, in_dim):
    out = W.shape[1]
    W3 = jnp.transpose(W.reshape(in_dim + RNN_UNITS, NM, out), (1, 0, 2))
    return W3[:, :in_dim, :], W3[:, in_dim:, :]


def _dup_gate(Wb):
    f = Wb.shape[0]
    W3 = Wb.reshape(f, 2, RNN_UNITS)
    eye = jnp.eye(2, dtype=Wb.dtype)
    return jnp.einsum('fgu,ab->afgbu', W3, eye).reshape(2 * f, GL)


def _dup_cand(Wb):
    f = Wb.shape[0]
    eye = jnp.eye(2, dtype=Wb.dtype)
    return jnp.einsum('fu,ab->afbu', Wb, eye).reshape(2 * f, HL)


def _cheb(S, x):
    c = x.shape[1]
    x3 = x.reshape(NP, NN, c)
    y1 = jax.lax.dot_general(
        S, x3, (((2,), (1,)), ((0,), (0,))),
        preferred_element_type=jnp.float32, precision=_PREC)
    y2 = 2.0 * jax.lax.dot_general(
        S, y1, (((2,), (1,)), ((0,), (0,))),
        preferred_element_type=jnp.float32, precision=_PREC) - x3
    return x, y1.reshape(NR, c), y2.reshape(NR, c)


def _dot(a, w):
    return jnp.dot(a, w, preferred_element_type=jnp.float32,
                   precision=_PREC)


def _cell(S, pair, h, W3k, Wr, bg, bc):
    q0, q1, q2 = _cheb(S, pair)
    acc = _dot(q0, W3k[0]) + _dot(q1, W3k[1]) + _dot(q2, W3k[2])
    gates = jax.nn.sigmoid(acc[:, :GL] + bg)
    r = gates[:, :HL]
    u = gates[:, HL:]
    r0, r1, r2 = _cheb(S, r * h)
    c = jnp.tanh(acc[:, GL:] + _dot(r0, Wr[0]) + _dot(r1, Wr[1])
                 + _dot(r2, Wr[2]) + bc)
    return u * h + (1.0 - u) * c


def _body(x_ref, s_ref, w0_ref, wr0_ref, bg0_ref, bc0_ref,
          w1_ref, wr1_ref, bg1_ref, bc1_ref, wfc_ref, bfc_ref,
          mask_ref, out_ref):
    S = jnp.broadcast_to(s_ref[...], (NP, NN, NN))
    w0 = [w0_ref[k] for k in range(NM)]
    wr0 = [wr0_ref[k] for k in range(NM)]
    w1 = [w1_ref[k] for k in range(NM)]
    wr1 = [wr1_ref[k] for k in range(NM)]
    bg0 = bg0_ref[...]
    bc0 = bc0_ref[...]
    bg1 = bg1_ref[...]
    bc1 = bc1_ref[...]

    def step(t, carry):
        h0, h1, last = carry
        pair0 = jnp.concatenate([h0, x_ref[t].T], axis=1)
        h0 = _cell(S, pair0, h0, w0, wr0, bg0, bc0)
        pair1 = jnp.concatenate([h0, h1], axis=1)
        h1 = _cell(S, pair1, h1, w1, wr1, bg1, bc1)
        m = mask_ref[t].reshape(NP, 1, HL)
        last = last + h1.reshape(NP, NN, HL) * m
        return h0, h1, last

    h0 = jnp.zeros((NR, HL), jnp.float32)
    h1 = jnp.zeros((NR, HL), jnp.float32)
    last = jnp.zeros((NP, NN, HL), jnp.float32)
    h0, h1, last = jax.lax.fori_loop(0, SEQ_LEN, step, (h0, h1, last))

    lr = jax.nn.relu(last.reshape(NR, HL))
    logits = _dot(lr, wfc_ref[...]) + bfc_ref[...]
    logits3 = logits.reshape(NP, NN, 2 * NUM_CLASSES)[:, :NUM_NODES, :]
    out_ref[...] = jnp.max(logits3, axis=1)


def kernel(input_seq, seq_lengths, supports, Wg0, bg0, Wc0, bc0,
           Wg1, bg1, Wc1, bc1, W_fc, b_fc):
    xseq = jnp.transpose(
        input_seq.reshape(NP, 2, SEQ_LEN, NUM_NODES, INPUT_DIM),
        (2, 1, 4, 0, 3))
    xseq = jnp.pad(xseq, ((0, 0), (0, 0), (0, 0), (0, 0),
                          (0, NN - NUM_NODES))
                   ).reshape(SEQ_LEN, 2 * INPUT_DIM, NR)
    S = jnp.pad(supports[0], ((0, NN - NUM_NODES), (0, NN - NUM_NODES)))

    wg0x, wg0h = _split_xh(Wg0, INPUT_DIM)
    wc0x, wc0h = _split_xh(Wc0, INPUT_DIM)
    wg1x, wg1h = _split_xh(Wg1, RNN_UNITS)
    wc1x, wc1h = _split_xh(Wc1, RNN_UNITS)

    w0, wr0, w1, wr1 = [], [], [], []
    for k in range(NM):
        top = jnp.concatenate(
            [_dup_gate(wg0h[k]), jnp.zeros((HL, HL), jnp.float32)], axis=1)
        bot = jnp.concatenate(
            [_dup_gate(wg0x[k]), _dup_cand(wc0x[k])], axis=1)
        w0.append(jnp.concatenate([top, bot], axis=0))
        wr0.append(_dup_cand(wc0h[k]))
        top = jnp.concatenate(
            [_dup_gate(wg1x[k]), _dup_cand(wc1x[k])], axis=1)
        bot = jnp.concatenate(
            [_dup_gate(wg1h[k]), jnp.zeros((HL, HL), jnp.float32)], axis=1)
        w1.append(jnp.concatenate([top, bot], axis=0))
        wr1.append(_dup_cand(wc1h[k]))
    w0 = jnp.stack(w0)
    wr0 = jnp.stack(wr0)
    w1 = jnp.stack(w1)
    wr1 = jnp.stack(wr1)

    def gate_bias(b):
        return jnp.broadcast_to(b.reshape(2, 1, RNN_UNITS),
                                (2, 2, RNN_UNITS)).reshape(1, GL)

    def cand_bias(b):
        return jnp.broadcast_to(b.reshape(1, RNN_UNITS),
                                (2, RNN_UNITS)).reshape(1, HL)

    eye = jnp.eye(2, dtype=W_fc.dtype)
    wfc2 = jnp.einsum('uc,ab->aubc', W_fc, eye).reshape(HL, 2 * NUM_CLASSES)
    bfc2 = jnp.broadcast_to(b_fc.reshape(1, NUM_CLASSES),
                            (2, NUM_CLASSES)).reshape(1, 2 * NUM_CLASSES)

    idx = jnp.clip(seq_lengths - 1, 0, SEQ_LEN - 1)
    onehot = (jnp.arange(SEQ_LEN)[:, None] == idx[None, :]).astype(jnp.float32)
    mask = jnp.repeat(onehot.reshape(SEQ_LEN, NP, 2, 1), RNN_UNITS,
                      axis=3).reshape(SEQ_LEN, NP, HL)

    pooled2 = pl.pallas_call(
        _body,
        out_shape=jax.ShapeDtypeStruct((NP, 2 * NUM_CLASSES), jnp.float32),
    )(xseq, S, w0, wr0, gate_bias(bg0), cand_bias(bc0),
      w1, wr1, gate_bias(bg1), cand_bias(bc1), wfc2, bfc2, mask)
    return pooled2.reshape(BATCH, NUM_CLASSES)
